# baseline (speedup 1.0000x reference)
import functools

import jax
import jax.numpy as jnp
from jax import lax
from jax.experimental import pallas as pl
from jax.experimental.pallas import tpu as pltpu
from jax.experimental.pallas import tpu_sc as plsc

_LF = 0.15875
_VGOAL = 0.9
_N = 100000
_NC = 2
_NS = 16
_NW = _NC * _NS
_L = 16
_RPW = _N // _NW
_WIN = 3136
_UNROLL = 14

_B = 2048
_G = (_N + _B - 1) // _B

_BIG = 3.4e38


def _extract_body(pk_ref, wt_ref, outx_ref, outy_ref, pvec_ref):
  outx_ref[...] = wt_ref[1, :]
  outy_ref[...] = wt_ref[2, :]

  @pl.when(pl.program_id(0) == 0)
  def _():
    i16 = lax.broadcasted_iota(jnp.int32, (_L,), 0)
    pvec_ref[...] = jnp.where(i16 == 0, pk_ref[0],
                              jnp.where(i16 == 1, pk_ref[1], 0.0))


_extract = pl.pallas_call(
    _extract_body,
    grid=(_G,),
    in_specs=[
        pl.BlockSpec(memory_space=pltpu.SMEM),
        pl.BlockSpec((6, _B), lambda g: (0, g)),
    ],
    out_shape=[
        jax.ShapeDtypeStruct((_N,), jnp.float32),
        jax.ShapeDtypeStruct((_N,), jnp.float32),
        jax.ShapeDtypeStruct((_L,), jnp.float32),
    ],
    out_specs=[
        pl.BlockSpec((_B,), lambda g: (g,)),
        pl.BlockSpec((_B,), lambda g: (g,)),
        pl.BlockSpec((_L,), lambda g: (0,)),
    ],
)


def _sc_search_body(wx_hbm, wy_hbm, pvec_hbm, out_hbm,
                    x_v, y_v, pv_v, outs_v, semx, semy):
  wid = lax.axis_index("s") * _NC + lax.axis_index("c")
  row0 = wid * _RPW
  base = pl.multiple_of(jnp.minimum((row0 // 8) * 8, _N - _WIN), 8)
  pltpu.sync_copy(pvec_hbm, pv_v)
  cx = pltpu.async_copy(wx_hbm.at[pl.ds(base, _WIN)], x_v, semx)
  cy = pltpu.async_copy(wy_hbm.at[pl.ds(base, _WIN)], y_v, semy)
  cx.wait()
  cy.wait()

  lanes = lax.broadcasted_iota(jnp.int32, (_L,), 0)
  zeros_i = jnp.zeros((_L,), jnp.int32)
  pv16 = pv_v[...]
  pcx = jnp.sum(jnp.where(lanes == 0, pv16, 0.0))
  pcy = jnp.sum(jnp.where(lanes == 1, pv16, 0.0))

  bd0 = jnp.full((_L,), _BIG, jnp.float32)

  @plsc.parallel_loop(0, _WIN, _L, unroll=_UNROLL, carry=(bd0, zeros_i))
  def _loop(i, carry):
    bd, br = carry
    xv = x_v[pl.ds(i, _L)]
    yv = y_v[pl.ds(i, _L)]
    dx = xv - pcx
    dy = yv - pcy
    d2 = dx * dx + dy * dy
    upd = d2 < bd
    return jnp.where(upd, d2, bd), jnp.where(upd, i + lanes, br)

  bd, br = _loop
  m = jnp.min(bd)
  r_win = jnp.min(jnp.where(bd == m, br, jnp.int32(0x7FFFFFFF)))
  g_row = (base + r_win).astype(jnp.float32)
  outvec = jnp.where(lanes == 0, m,
                     jnp.where(lanes == 1, g_row, 0.0))
  outs_v[...] = outvec
  pltpu.sync_copy(outs_v, out_hbm.at[wid])


@functools.cache
def _get_sc_search():
  return pl.kernel(
      _sc_search_body,
      out_type=jax.ShapeDtypeStruct((_NW, _L), jnp.float32),
      mesh=plsc.VectorSubcoreMesh(core_axis_name="c", subcore_axis_name="s",
                                  num_cores=_NC, num_subcores=_NS),
      compiler_params=pltpu.CompilerParams(needs_layout_passes=False),
      scratch_types=[
          pltpu.VMEM((_WIN,), jnp.float32),
          pltpu.VMEM((_WIN,), jnp.float32),
          pltpu.VMEM((_L,), jnp.float32),
          pltpu.VMEM((_L,), jnp.float32),
          pltpu.SemaphoreType.DMA,
          pltpu.SemaphoreType.DMA,
      ],
  )


def _atan(u):
  t = jnp.abs(u)
  inv = t > 1.0
  z = jnp.where(inv, 1.0 / jnp.maximum(t, 1e-30), t)
  big = z > 0.4142135623730951
  z2 = jnp.where(big, (z - 1.0) / (z + 1.0), z)
  w = z2 * z2
  p = ((8.05374449538e-2 * w - 1.38776856032e-1) * w
       + 1.99777106478e-1) * w - 3.33329491539e-1
  r = z2 + z2 * w * p
  r = jnp.where(big, jnp.float32(0.7853981633974483) + r, r)
  r = jnp.where(inv, jnp.float32(1.5707963267948966) - r, r)
  return jnp.where(u < 0.0, -r, r)


def _sincos(t):
  tt = t * jnp.float32(0.6366197723675814)
  n = (tt + jnp.float32(0.5) * jnp.sign(tt)).astype(jnp.int32)
  nf = n.astype(jnp.float32)
  r = (t - nf * jnp.float32(1.5707962513e+00)) - nf * jnp.float32(7.5497894159e-08)
  q = n & 3
  r2 = r * r
  sp = r + r * r2 * (jnp.float32(-1.6666654611e-01)
                     + r2 * (jnp.float32(8.3321608736e-03)
                             + r2 * jnp.float32(-1.9515295891e-04)))
  cp = 1.0 + r2 * (jnp.float32(-0.5)
                   + r2 * (jnp.float32(4.166664568298827e-02)
                           + r2 * (jnp.float32(-1.388731625493765e-03)
                                   + r2 * jnp.float32(2.443315711809948e-05))))
  s = jnp.where(q == 0, sp, jnp.where(q == 1, cp,
                                      jnp.where(q == 2, -sp, -cp)))
  c = jnp.where(q == 0, cp, jnp.where(q == 1, -sp,
                                      jnp.where(q == 2, -cp, sp)))
  return s, c


def _merge_body(pk_ref, pose_ref, cand_ref, wt_ref,
                steer_ref, v_ref, ce_ref, he_ref, row_v, sem):
  cand = cand_ref[...]
  lane = lax.broadcasted_iota(jnp.int32, (_NW, _L), 1)
  wrow = lax.broadcasted_iota(jnp.int32, (_NW, 1), 0).astype(jnp.float32)
  d2f = jnp.where(lane == 0, cand, _BIG)
  m = jnp.min(d2f)
  rowd2 = jnp.min(d2f, axis=1, keepdims=True)
  rowidx = jnp.max(jnp.where(lane == 1, cand, -_BIG), axis=1, keepdims=True)
  key = rowidx * jnp.float32(_NW) + wrow
  keysel = jnp.where(rowd2 <= m, key, _BIG)
  kstar = jnp.min(keysel)
  istar = jnp.floor(kstar / jnp.float32(_NW)).astype(jnp.int32)

  c0 = pl.multiple_of((istar // 128) * 128, 128)
  cp = pltpu.make_async_copy(wt_ref.at[:, pl.ds(c0, 128)], row_v, sem)
  cp.start()
  cp.wait()
  j = istar - c0
  li = lax.broadcasted_iota(jnp.int32, (6, 128), 1)
  ri = lax.broadcasted_iota(jnp.int32, (6, 128), 0)
  rv = row_v[...]
  selc = li == j
  wx = jnp.sum(jnp.where(selc & (ri == 1), rv, 0.0))
  wy = jnp.sum(jnp.where(selc & (ri == 2), rv, 0.0))
  wh = jnp.sum(jnp.where(selc & (ri == 3), rv, 0.0))
  ws = jnp.sum(jnp.where(selc & (ri == 5), rv, 0.0))

  pcx = pk_ref[0]
  pcy = pk_ref[1]
  k_e = pk_ref[2]
  k_h = pk_ref[3]
  theta = pose_ref[2]

  pi = jnp.float32(jnp.pi)
  thetap = jnp.remainder(theta + pi, 2.0 * pi)
  s, c = _sincos(thetap + pi / 2.0)
  fav0 = -c
  fav1 = -s
  ce = (pcx - wx) * fav0 + (pcy - wy) * fav1
  he = jnp.remainder(wh - thetap + pi, 2.0 * pi) - pi
  v = ws * jnp.float32(_VGOAL)
  steer = k_h * he + _atan(k_e * -ce / (v + 1e-05))
  steer_ref[0, 0] = steer
  v_ref[0, 0] = v
  ce_ref[0, 0] = ce
  he_ref[0, 0] = he


_merge = pl.pallas_call(
    _merge_body,
    in_specs=[
        pl.BlockSpec(memory_space=pltpu.SMEM),
        pl.BlockSpec(memory_space=pltpu.SMEM),
        pl.BlockSpec(memory_space=pltpu.VMEM),
        pl.BlockSpec(memory_space=pltpu.MemorySpace.HBM),
    ],
    out_shape=[jax.ShapeDtypeStruct((1, 1), jnp.float32)] * 4,
    out_specs=[pl.BlockSpec(memory_space=pltpu.SMEM)] * 4,
    scratch_shapes=[
        pltpu.VMEM((6, 128), jnp.float32),
        pltpu.SemaphoreType.DMA,
    ],
)


def kernel(pose, waypoints, k_e, k_h):
  s2 = jnp.sin(pose[2])
  c2 = jnp.cos(pose[2])
  pk = jnp.stack([pose[0] + _LF * s2, pose[1] + _LF * c2,
                  k_e.astype(jnp.float32), k_h.astype(jnp.float32)])
  wt = waypoints.T
  wx, wy, pvec16 = _extract(pk, wt)
  cand = _get_sc_search()(wx, wy, pvec16)
  steer, v, ce, he = _merge(pk, pose, cand, wt)
  return (steer[0, 0], v[0, 0], ce[0, 0], he[0, 0])


# speedup vs baseline: 1.8481x; 1.8481x over previous
import functools

import jax
import jax.numpy as jnp
from jax import lax
from jax.experimental import pallas as pl
from jax.experimental.pallas import tpu as pltpu
from jax.experimental.pallas import tpu_sc as plsc

_LF = 0.15875
_VGOAL = 0.9
_N = 100000
_NC = 2
_NS = 16
_NW = _NC * _NS
_L = 16
_RPW = _N // _NW
_TWIN = 3328
_UNROLL = 13
_TAIL0 = 99968
_TAILN = _N - _TAIL0

_BIG = 3.4e38


def _sc_search_body(wt_hbm, pvec_hbm, out_hbm, buf_v, pv_v, outs_v, semb):
  wid = lax.axis_index("s") * _NC + lax.axis_index("c")
  row0 = wid * _RPW
  base = pl.multiple_of(jnp.minimum((row0 // 128) * 128, _N - _TWIN), 128)
  pltpu.sync_copy(pvec_hbm, pv_v)
  cb = pltpu.async_copy(wt_hbm.at[:, pl.ds(base, _TWIN)], buf_v, semb)
  cb.wait()

  lanes = lax.broadcasted_iota(jnp.int32, (_L,), 0)
  zeros_i = jnp.zeros((_L,), jnp.int32)
  pv16 = pv_v[...]
  pcx = jnp.sum(jnp.where(lanes == 0, pv16, 0.0))
  pcy = jnp.sum(jnp.where(lanes == 1, pv16, 0.0))

  def step(i, bd, br):
    xv = buf_v[1, pl.ds(i, _L)]
    yv = buf_v[2, pl.ds(i, _L)]
    dx = xv - pcx
    dy = yv - pcy
    d2 = dx * dx + dy * dy
    upd = d2 < bd
    return jnp.where(upd, d2, bd), jnp.where(upd, i + lanes, br)

  bd0 = jnp.full((_L,), _BIG, jnp.float32)

  @plsc.parallel_loop(0, _TWIN, _L, unroll=_UNROLL, carry=(bd0, zeros_i))
  def _loop(i, carry):
    bd, br = carry
    return step(i, bd, br)

  bd, br = _loop

  m = jnp.min(bd)
  c_win = jnp.min(jnp.where(bd == m, br, jnp.int32(0x7FFFFFFF)))
  g_row = (base + c_win).astype(jnp.float32)
  co = pl.multiple_of((c_win // _L) * _L, _L)
  jsel = lanes == (c_win - co)
  xw = jnp.sum(jnp.where(jsel, buf_v[1, pl.ds(co, _L)], 0.0))
  yw = jnp.sum(jnp.where(jsel, buf_v[2, pl.ds(co, _L)], 0.0))
  hw = jnp.sum(jnp.where(jsel, buf_v[3, pl.ds(co, _L)], 0.0))
  sw = jnp.sum(jnp.where(jsel, buf_v[5, pl.ds(co, _L)], 0.0))
  outvec = jnp.where(lanes == 0, m,
                     jnp.where(lanes == 1, g_row,
                               jnp.where(lanes == 3, xw,
                                         jnp.where(lanes == 4, yw,
                                                   jnp.where(lanes == 5, hw,
                                                             sw)))))
  outs_v[...] = outvec
  pltpu.sync_copy(outs_v, out_hbm.at[wid])


@functools.cache
def _get_sc_search():
  return pl.kernel(
      _sc_search_body,
      out_type=jax.ShapeDtypeStruct((_NW, _L), jnp.float32),
      mesh=plsc.VectorSubcoreMesh(core_axis_name="c", subcore_axis_name="s",
                                  num_cores=_NC, num_subcores=_NS),
      compiler_params=pltpu.CompilerParams(needs_layout_passes=False,
                                           use_tc_tiling_on_sc=True),
      scratch_types=[
          pltpu.VMEM((6, _TWIN), jnp.float32),
          pltpu.VMEM((_L,), jnp.float32),
          pltpu.VMEM((_L,), jnp.float32),
          pltpu.SemaphoreType.DMA,
      ],
  )


def _atan(u):
  t = jnp.abs(u)
  inv = t > 1.0
  z = jnp.where(inv, 1.0 / jnp.maximum(t, 1e-30), t)
  big = z > 0.4142135623730951
  z2 = jnp.where(big, (z - 1.0) / (z + 1.0), z)
  w = z2 * z2
  p = ((8.05374449538e-2 * w - 1.38776856032e-1) * w
       + 1.99777106478e-1) * w - 3.33329491539e-1
  r = z2 + z2 * w * p
  r = jnp.where(big, jnp.float32(0.7853981633974483) + r, r)
  r = jnp.where(inv, jnp.float32(1.5707963267948966) - r, r)
  return jnp.where(u < 0.0, -r, r)


def _sincos(t):
  tt = t * jnp.float32(0.6366197723675814)
  n = (tt + jnp.float32(0.5) * jnp.sign(tt)).astype(jnp.int32)
  nf = n.astype(jnp.float32)
  r = (t - nf * jnp.float32(1.5707962513e+00)) - nf * jnp.float32(7.5497894159e-08)
  q = n & 3
  r2 = r * r
  sp = r + r * r2 * (jnp.float32(-1.6666654611e-01)
                     + r2 * (jnp.float32(8.3321608736e-03)
                             + r2 * jnp.float32(-1.9515295891e-04)))
  cp = 1.0 + r2 * (jnp.float32(-0.5)
                   + r2 * (jnp.float32(4.166664568298827e-02)
                           + r2 * (jnp.float32(-1.388731625493765e-03)
                                   + r2 * jnp.float32(2.443315711809948e-05))))
  s = jnp.where(q == 0, sp, jnp.where(q == 1, cp,
                                      jnp.where(q == 2, -sp, -cp)))
  c = jnp.where(q == 0, cp, jnp.where(q == 1, -sp,
                                      jnp.where(q == 2, -cp, sp)))
  return s, c


def _merge_body(pk_ref, pose_ref, cand_ref, tail_ref,
                steer_ref, v_ref, ce_ref, he_ref):
  cand = cand_ref[...]
  lane = lax.broadcasted_iota(jnp.int32, (_NW, _L), 1)
  wrow = lax.broadcasted_iota(jnp.int32, (_NW, 1), 0).astype(jnp.float32)
  d2f = jnp.where(lane == 0, cand, _BIG)
  m = jnp.min(d2f)
  rowd2 = jnp.min(d2f, axis=1, keepdims=True)
  rowidx = jnp.max(jnp.where(lane == 1, cand, -_BIG), axis=1, keepdims=True)
  key = rowidx * jnp.float32(_NW) + wrow
  keysel = jnp.where(rowd2 <= m, key, _BIG)
  kstar = jnp.min(keysel)
  sel = keysel <= kstar
  row = jnp.sum(jnp.where(sel, cand, 0.0), axis=0, keepdims=True)
  lane1 = lax.broadcasted_iota(jnp.int32, (1, _L), 1)
  wx = jnp.sum(jnp.where(lane1 == 3, row, 0.0))
  wy = jnp.sum(jnp.where(lane1 == 4, row, 0.0))
  wh = jnp.sum(jnp.where(lane1 == 5, row, 0.0))
  ws = jnp.sum(jnp.where(lane1 == 7, row, 0.0))

  pcx = pk_ref[0]
  pcy = pk_ref[1]

  blk = tail_ref[...]
  li6 = lax.broadcasted_iota(jnp.int32, (6, 128), 1)
  ri6 = lax.broadcasted_iota(jnp.int32, (6, 128), 0)
  xv = jnp.sum(jnp.where(ri6 == 1, blk, 0.0), axis=0, keepdims=True)
  yv = jnp.sum(jnp.where(ri6 == 2, blk, 0.0), axis=0, keepdims=True)
  li1 = lax.broadcasted_iota(jnp.int32, (1, 128), 1)
  d2v = (xv - pcx) * (xv - pcx) + (yv - pcy) * (yv - pcy)
  d2v = jnp.where(li1 < _TAILN, d2v, _BIG)
  d2t = jnp.min(d2v)
  ct = jnp.min(jnp.where(d2v <= d2t, li1, jnp.int32(0x7FFFFFFF)))
  tsel = (li6 == ct)
  xt = jnp.sum(jnp.where(tsel & (ri6 == 1), blk, 0.0))
  yt = jnp.sum(jnp.where(tsel & (ri6 == 2), blk, 0.0))
  ht = jnp.sum(jnp.where(tsel & (ri6 == 3), blk, 0.0))
  st = jnp.sum(jnp.where(tsel & (ri6 == 5), blk, 0.0))
  use_t = d2t < m
  wx = jnp.where(use_t, xt, wx)
  wy = jnp.where(use_t, yt, wy)
  wh = jnp.where(use_t, ht, wh)
  ws = jnp.where(use_t, st, ws)
  k_e = pk_ref[2]
  k_h = pk_ref[3]
  theta = pose_ref[2]

  pi = jnp.float32(jnp.pi)
  thetap = jnp.remainder(theta + pi, 2.0 * pi)
  s, c = _sincos(thetap + pi / 2.0)
  fav0 = -c
  fav1 = -s
  ce = (pcx - wx) * fav0 + (pcy - wy) * fav1
  he = jnp.remainder(wh - thetap + pi, 2.0 * pi) - pi
  v = ws * jnp.float32(_VGOAL)
  steer = k_h * he + _atan(k_e * -ce / (v + 1e-05))
  steer_ref[0, 0] = steer
  v_ref[0, 0] = v
  ce_ref[0, 0] = ce
  he_ref[0, 0] = he


_merge = pl.pallas_call(
    _merge_body,
    grid=(1,),
    in_specs=[
        pl.BlockSpec(memory_space=pltpu.SMEM),
        pl.BlockSpec(memory_space=pltpu.SMEM),
        pl.BlockSpec((_NW, _L), lambda i: (0, 0)),
        pl.BlockSpec((6, 128), lambda i: (0, _TAIL0 // 128)),
    ],
    out_shape=[jax.ShapeDtypeStruct((1, 1), jnp.float32)] * 4,
    out_specs=[pl.BlockSpec((1, 1), lambda i: (0, 0),
                            memory_space=pltpu.SMEM)] * 4,
)


def kernel(pose, waypoints, k_e, k_h):
  s2 = jnp.sin(pose[2])
  c2 = jnp.cos(pose[2])
  pk = jnp.stack([pose[0] + _LF * s2, pose[1] + _LF * c2,
                  k_e.astype(jnp.float32), k_h.astype(jnp.float32)])
  pvec16 = jnp.pad(pk, (0, _L - 4))
  wt = waypoints.T
  cand = _get_sc_search()(wt, pvec16)
  steer, v, ce, he = _merge(pk, pose, cand, wt)
  return (steer[0, 0], v[0, 0], ce[0, 0], he[0, 0])


# speedup vs baseline: 1.9383x; 1.0489x over previous
import functools

import jax
import jax.numpy as jnp
from jax import lax
from jax.experimental import pallas as pl
from jax.experimental.pallas import tpu as pltpu
from jax.experimental.pallas import tpu_sc as plsc

_LF = 0.15875
_VGOAL = 0.9
_N = 100000
_NC = 2
_NS = 16
_NW = _NC * _NS
_L = 16
_RPW = _N // _NW
_TWIN = 3328
_UNROLL = 13
_TAIL0 = 99968
_TAILN = _N - _TAIL0

_BIG = 3.4e38


def _sc_search_body(wt_hbm, pvec_hbm, out_hbm, buf_v, pv_v, outs_v,
                    semb, semb2):
  wid = lax.axis_index("s") * _NC + lax.axis_index("c")
  row0 = wid * _RPW
  base = pl.multiple_of(jnp.minimum((row0 // 128) * 128, _N - _TWIN), 128)
  _H = _TWIN // 2
  cb = pltpu.async_copy(wt_hbm.at[:, pl.ds(base, _H)],
                        buf_v.at[:, pl.ds(0, _H)], semb)
  cb2 = pltpu.async_copy(wt_hbm.at[:, pl.ds(base + _H, _H)],
                         buf_v.at[:, pl.ds(_H, _H)], semb2)
  pltpu.sync_copy(pvec_hbm, pv_v.at[pl.ds(0, 4)])

  lanes = lax.broadcasted_iota(jnp.int32, (_L,), 0)
  zeros_i = jnp.zeros((_L,), jnp.int32)
  pv16 = pv_v[...]
  pcx = jnp.sum(jnp.where(lanes == 0, pv16, 0.0))
  pcy = jnp.sum(jnp.where(lanes == 1, pv16, 0.0))

  def step(i, bd, br):
    xv = buf_v[1, pl.ds(i, _L)]
    yv = buf_v[2, pl.ds(i, _L)]
    dx = xv - pcx
    dy = yv - pcy
    d2 = dx * dx + dy * dy
    upd = d2 < bd
    return jnp.where(upd, d2, bd), jnp.where(upd, i + lanes, br)

  bd0 = jnp.full((_L,), _BIG, jnp.float32)
  cb.wait()

  @plsc.parallel_loop(0, _H, _L, unroll=_UNROLL, carry=(bd0, zeros_i))
  def _loop(i, carry):
    bd, br = carry
    return step(i, bd, br)

  bd, br = _loop
  cb2.wait()

  @plsc.parallel_loop(_H, _TWIN, _L, unroll=_UNROLL, carry=(bd, br))
  def _loop2(i, carry):
    bd, br = carry
    return step(i, bd, br)

  bd, br = _loop2

  m = jnp.min(bd)
  c_win = jnp.min(jnp.where(bd == m, br, jnp.int32(0x7FFFFFFF)))
  g_row = (base + c_win).astype(jnp.float32)
  co = pl.multiple_of((c_win // _L) * _L, _L)
  jsel = lanes == (c_win - co)
  xw = jnp.sum(jnp.where(jsel, buf_v[1, pl.ds(co, _L)], 0.0))
  yw = jnp.sum(jnp.where(jsel, buf_v[2, pl.ds(co, _L)], 0.0))
  hw = jnp.sum(jnp.where(jsel, buf_v[3, pl.ds(co, _L)], 0.0))
  sw = jnp.sum(jnp.where(jsel, buf_v[5, pl.ds(co, _L)], 0.0))
  outvec = jnp.where(lanes == 0, m,
                     jnp.where(lanes == 1, g_row,
                               jnp.where(lanes == 3, xw,
                                         jnp.where(lanes == 4, yw,
                                                   jnp.where(lanes == 5, hw,
                                                             sw)))))
  outs_v[...] = outvec
  pltpu.sync_copy(outs_v, out_hbm.at[wid])


@functools.cache
def _get_sc_search():
  return pl.kernel(
      _sc_search_body,
      out_type=jax.ShapeDtypeStruct((_NW, _L), jnp.float32),
      mesh=plsc.VectorSubcoreMesh(core_axis_name="c", subcore_axis_name="s",
                                  num_cores=_NC, num_subcores=_NS),
      compiler_params=pltpu.CompilerParams(needs_layout_passes=False,
                                           use_tc_tiling_on_sc=True),
      scratch_types=[
          pltpu.VMEM((6, _TWIN), jnp.float32),
          pltpu.VMEM((_L,), jnp.float32),
          pltpu.VMEM((_L,), jnp.float32),
          pltpu.SemaphoreType.DMA,
          pltpu.SemaphoreType.DMA,
      ],
  )


def _atan(u):
  t = jnp.abs(u)
  inv = t > 1.0
  z = jnp.where(inv, 1.0 / jnp.maximum(t, 1e-30), t)
  big = z > 0.4142135623730951
  z2 = jnp.where(big, (z - 1.0) / (z + 1.0), z)
  w = z2 * z2
  p = ((8.05374449538e-2 * w - 1.38776856032e-1) * w
       + 1.99777106478e-1) * w - 3.33329491539e-1
  r = z2 + z2 * w * p
  r = jnp.where(big, jnp.float32(0.7853981633974483) + r, r)
  r = jnp.where(inv, jnp.float32(1.5707963267948966) - r, r)
  return jnp.where(u < 0.0, -r, r)


def _sincos(t):
  tt = t * jnp.float32(0.6366197723675814)
  n = (tt + jnp.float32(0.5) * jnp.sign(tt)).astype(jnp.int32)
  nf = n.astype(jnp.float32)
  r = (t - nf * jnp.float32(1.5707962513e+00)) - nf * jnp.float32(7.5497894159e-08)
  q = n & 3
  r2 = r * r
  sp = r + r * r2 * (jnp.float32(-1.6666654611e-01)
                     + r2 * (jnp.float32(8.3321608736e-03)
                             + r2 * jnp.float32(-1.9515295891e-04)))
  cp = 1.0 + r2 * (jnp.float32(-0.5)
                   + r2 * (jnp.float32(4.166664568298827e-02)
                           + r2 * (jnp.float32(-1.388731625493765e-03)
                                   + r2 * jnp.float32(2.443315711809948e-05))))
  s = jnp.where(q == 0, sp, jnp.where(q == 1, cp,
                                      jnp.where(q == 2, -sp, -cp)))
  c = jnp.where(q == 0, cp, jnp.where(q == 1, -sp,
                                      jnp.where(q == 2, -cp, sp)))
  return s, c


def _merge_body(pk_ref, pose_ref, cand_ref, tail_ref,
                steer_ref, v_ref, ce_ref, he_ref):
  cand = cand_ref[...]
  lane = lax.broadcasted_iota(jnp.int32, (_NW, _L), 1)
  wrow = lax.broadcasted_iota(jnp.int32, (_NW, 1), 0).astype(jnp.float32)
  d2f = jnp.where(lane == 0, cand, _BIG)
  m = jnp.min(d2f)
  rowd2 = jnp.min(d2f, axis=1, keepdims=True)
  rowidx = jnp.max(jnp.where(lane == 1, cand, -_BIG), axis=1, keepdims=True)
  key = rowidx * jnp.float32(_NW) + wrow
  keysel = jnp.where(rowd2 <= m, key, _BIG)
  kstar = jnp.min(keysel)
  sel = keysel <= kstar
  row = jnp.sum(jnp.where(sel, cand, 0.0), axis=0, keepdims=True)
  lane1 = lax.broadcasted_iota(jnp.int32, (1, _L), 1)
  wx = jnp.sum(jnp.where(lane1 == 3, row, 0.0))
  wy = jnp.sum(jnp.where(lane1 == 4, row, 0.0))
  wh = jnp.sum(jnp.where(lane1 == 5, row, 0.0))
  ws = jnp.sum(jnp.where(lane1 == 7, row, 0.0))

  pcx = pk_ref[0]
  pcy = pk_ref[1]

  blk = tail_ref[...]
  li6 = lax.broadcasted_iota(jnp.int32, (6, 128), 1)
  ri6 = lax.broadcasted_iota(jnp.int32, (6, 128), 0)
  xv = jnp.sum(jnp.where(ri6 == 1, blk, 0.0), axis=0, keepdims=True)
  yv = jnp.sum(jnp.where(ri6 == 2, blk, 0.0), axis=0, keepdims=True)
  li1 = lax.broadcasted_iota(jnp.int32, (1, 128), 1)
  d2v = (xv - pcx) * (xv - pcx) + (yv - pcy) * (yv - pcy)
  d2v = jnp.where(li1 < _TAILN, d2v, _BIG)
  d2t = jnp.min(d2v)
  ct = jnp.min(jnp.where(d2v <= d2t, li1, jnp.int32(0x7FFFFFFF)))
  tsel = (li6 == ct)
  xt = jnp.sum(jnp.where(tsel & (ri6 == 1), blk, 0.0))
  yt = jnp.sum(jnp.where(tsel & (ri6 == 2), blk, 0.0))
  ht = jnp.sum(jnp.where(tsel & (ri6 == 3), blk, 0.0))
  st = jnp.sum(jnp.where(tsel & (ri6 == 5), blk, 0.0))
  use_t = d2t < m
  wx = jnp.where(use_t, xt, wx)
  wy = jnp.where(use_t, yt, wy)
  wh = jnp.where(use_t, ht, wh)
  ws = jnp.where(use_t, st, ws)
  k_e = pk_ref[2]
  k_h = pk_ref[3]
  theta = pose_ref[2]

  pi = jnp.float32(jnp.pi)
  thetap = jnp.remainder(theta + pi, 2.0 * pi)
  s, c = _sincos(thetap + pi / 2.0)
  fav0 = -c
  fav1 = -s
  ce = (pcx - wx) * fav0 + (pcy - wy) * fav1
  he = jnp.remainder(wh - thetap + pi, 2.0 * pi) - pi
  v = ws * jnp.float32(_VGOAL)
  steer = k_h * he + _atan(k_e * -ce / (v + 1e-05))
  steer_ref[0, 0] = steer
  v_ref[0, 0] = v
  ce_ref[0, 0] = ce
  he_ref[0, 0] = he


_merge = pl.pallas_call(
    _merge_body,
    grid=(1,),
    in_specs=[
        pl.BlockSpec(memory_space=pltpu.SMEM),
        pl.BlockSpec(memory_space=pltpu.SMEM),
        pl.BlockSpec((_NW, _L), lambda i: (0, 0)),
        pl.BlockSpec((6, 128), lambda i: (0, _TAIL0 // 128)),
    ],
    out_shape=[jax.ShapeDtypeStruct((1, 1), jnp.float32)] * 4,
    out_specs=[pl.BlockSpec((1, 1), lambda i: (0, 0),
                            memory_space=pltpu.SMEM)] * 4,
)


def kernel(pose, waypoints, k_e, k_h):
  s2 = jnp.sin(pose[2])
  c2 = jnp.cos(pose[2])
  pk = jnp.stack([pose[0] + _LF * s2, pose[1] + _LF * c2,
                  k_e.astype(jnp.float32), k_h.astype(jnp.float32)])
  wt = waypoints.T
  cand = _get_sc_search()(wt, pk)
  steer, v, ce, he = _merge(pk, pose, cand, wt)
  return (steer[0, 0], v[0, 0], ce[0, 0], he[0, 0])
